# baseline (speedup 1.0000x reference)
import jax
import jax.numpy as jnp
from jax.experimental import pallas as pl


_ROWS = 4096
_SEQ = 50
_NUM = 256
_BLOCK_R = 512


def _onehot_block(idx_ref, out_ref):
    idx = idx_ref[...]
    iota = jax.lax.broadcasted_iota(jnp.int32, (_SEQ, _BLOCK_R, _NUM), 2)
    out_ref[...] = (idx[:, :, None] == iota).astype(jnp.float32)


def kernel(input_tensor):
    idx_t = input_tensor.T
    out_t = pl.pallas_call(
        _onehot_block,
        grid=(_ROWS // _BLOCK_R,),
        in_specs=[pl.BlockSpec((_SEQ, _BLOCK_R), lambda i: (0, i))],
        out_specs=pl.BlockSpec((_SEQ, _BLOCK_R, _NUM), lambda i: (0, i, 0)),
        out_shape=jax.ShapeDtypeStruct((_SEQ, _ROWS, _NUM), jnp.float32),
    )(idx_t)
    return out_t.transpose(1, 0, 2)


# speedup vs baseline: 1.0315x; 1.0315x over previous
import jax
import jax.numpy as jnp
from jax.experimental import pallas as pl
from jax.experimental.pallas import tpu as pltpu


_ROWS = 4096
_SEQ = 50
_NUM = 256
_CHUNK_R = 256
_NCHUNK = _ROWS // _CHUNK_R
_K = 3


def _onehot_manual(idx_ref, out_ref, buf, sem):
    def copy(i):
        slot = i % _K
        return pltpu.make_async_copy(
            buf.at[slot],
            out_ref.at[:, pl.ds(i * _CHUNK_R, _CHUNK_R)],
            sem.at[slot],
        )

    for i in range(_NCHUNK):
        slot = i % _K
        if i >= _K:
            copy(i - _K).wait()
        idx = idx_ref[:, pl.ds(i * _CHUNK_R, _CHUNK_R)]
        iota = jax.lax.broadcasted_iota(jnp.int32, (_SEQ, _CHUNK_R, _NUM), 2)
        buf[slot] = (idx[:, :, None] == iota).astype(jnp.float32)
        copy(i).start(priority=i % 2)
    for i in range(_NCHUNK - _K, _NCHUNK):
        copy(i).wait()


def kernel(input_tensor):
    idx_t = input_tensor.T
    out_t = pl.pallas_call(
        _onehot_manual,
        in_specs=[pl.BlockSpec(memory_space=pltpu.VMEM)],
        out_specs=pl.BlockSpec(memory_space=pltpu.HBM),
        out_shape=jax.ShapeDtypeStruct((_SEQ, _ROWS, _NUM), jnp.float32),
        scratch_shapes=[
            pltpu.VMEM((_K, _SEQ, _CHUNK_R, _NUM), jnp.float32),
            pltpu.SemaphoreType.DMA((_K,)),
        ],
    )(idx_t)
    return out_t.transpose(1, 0, 2)


# speedup vs baseline: 1.0355x; 1.0038x over previous
import jax
import jax.numpy as jnp
from jax.experimental import pallas as pl
from jax.experimental.pallas import tpu as pltpu


_ROWS = 4096
_SEQ = 50
_NUM = 256
_BLOCK_R = 256


def _onehot_block(idx_ref, out_ref):
    i = pl.program_id(0)
    idx = idx_ref[:, pl.ds(i * _BLOCK_R, _BLOCK_R)]
    iota = jax.lax.broadcasted_iota(jnp.int32, (_SEQ, _BLOCK_R, _NUM), 2)
    out_ref[...] = (idx[:, :, None] == iota).astype(jnp.float32)


def kernel(input_tensor):
    idx_t = input_tensor.T
    out_t = pl.pallas_call(
        _onehot_block,
        grid=(_ROWS // _BLOCK_R,),
        in_specs=[pl.BlockSpec(memory_space=pltpu.VMEM)],
        out_specs=pl.BlockSpec((_SEQ, _BLOCK_R, _NUM), lambda i: (0, i, 0)),
        out_shape=jax.ShapeDtypeStruct((_SEQ, _ROWS, _NUM), jnp.float32),
    )(idx_t)
    return out_t.transpose(1, 0, 2)


# speedup vs baseline: 1.0533x; 1.0172x over previous
import jax
import jax.numpy as jnp
from jax.experimental import pallas as pl
from jax.experimental.pallas import tpu as pltpu


_ROWS = 4096
_SEQ = 50
_NUM = 256
_CHUNK = 256
_K = 3

_SCHED = [(0, 32), (32, 32), (64, 64), (128, 128)]
_SCHED += [(s, _CHUNK) for s in range(_CHUNK, _ROWS, _CHUNK)]


def _onehot_manual(idx_ref, out_ref, buf, sem):
    def copy(j):
        start, size = _SCHED[j]
        slot = j % _K
        return pltpu.make_async_copy(
            buf.at[slot, :, pl.ds(0, size)],
            out_ref.at[:, pl.ds(start, size)],
            sem.at[slot],
        )

    for j, (start, size) in enumerate(_SCHED):
        slot = j % _K
        if j >= _K:
            copy(j - _K).wait()
        idx = idx_ref[:, pl.ds(start, size)]
        iota = jax.lax.broadcasted_iota(jnp.int32, (_SEQ, size, _NUM), 2)
        buf[slot, :, pl.ds(0, size)] = (idx[:, :, None] == iota).astype(
            jnp.float32
        )
        copy(j).start()
    for j in range(len(_SCHED) - _K, len(_SCHED)):
        copy(j).wait()


def kernel(input_tensor):
    idx_t = input_tensor.T
    out_t = pl.pallas_call(
        _onehot_manual,
        in_specs=[pl.BlockSpec(memory_space=pltpu.VMEM)],
        out_specs=pl.BlockSpec(memory_space=pltpu.HBM),
        out_shape=jax.ShapeDtypeStruct((_SEQ, _ROWS, _NUM), jnp.float32),
        scratch_shapes=[
            pltpu.VMEM((_K, _SEQ, _CHUNK, _NUM), jnp.float32),
            pltpu.SemaphoreType.DMA((_K,)),
        ],
    )(idx_t)
    return out_t.transpose(1, 0, 2)
